# baseline (speedup 1.0000x reference)
import functools

import jax
import jax.numpy as jnp
from jax import lax
from jax.experimental import pallas as pl
from jax.experimental.pallas import tpu as pltpu
from jax.experimental.pallas import tpu_sc as plsc

_D = 64
_B = 4096
_H = 200
_V = 1000000
_L = 128
_NW = 32

_mesh = plsc.VectorSubcoreMesh(core_axis_name="c", subcore_axis_name="s")


@functools.partial(
    pl.kernel,
    out_type=jax.ShapeDtypeStruct((_H, _D, _B), jnp.float32),
    mesh=_mesh,
    scratch_types=[
        pltpu.VMEM((_H, _L), jnp.int32),
        pltpu.VMEM((_L,), jnp.int32),
        pltpu.VMEM((_L,), jnp.int32),
        pltpu.VMEM((_L,), jnp.int32),
        pltpu.VMEM((_L,), jnp.int32),
        pltpu.VMEM((_L, _L), jnp.float32),
        pltpu.VMEM((_L, _L), jnp.float32),
        pltpu.VMEM((_D, _L), jnp.float32),
        pltpu.VMEM((_D, _L), jnp.float32),
        pltpu.SemaphoreType.DMA,
        pltpu.SemaphoreType.DMA,
        pltpu.SemaphoreType.DMA,
        pltpu.SemaphoreType.DMA,
    ],
    compiler_params=pltpu.CompilerParams(
        use_tc_tiling_on_sc=True, needs_layout_passes=False),
)
def _gather_t(idx_hbm, tab_hbm, out_hbm, idx_v, i20, i21, hb0, hb1,
              rows0, rows1, til0, til1, sg0, sg1, sw0, sw1):
    wid = lax.axis_index("s") * 2 + lax.axis_index("c")
    b0 = wid * _L

    pltpu.sync_copy(idx_hbm.at[:, pl.ds(b0, _L)], idx_v)

    i2s = (i20, i21)
    hbs = (hb0, hb1)
    rows = (rows0, rows1)
    tils = (til0, til1)
    sgs = (sg0, sg1)
    sws = (sw0, sw1)

    lane = lax.iota(jnp.int32, 16)
    bvecs = [lane + (c * 16) for c in range(8)]

    def prep(h, p):
        for c in range(8):
            iv = idx_v[h, pl.ds(c * 16, 16)]
            i2s[p][pl.ds(c * 16, 16)] = lax.shift_right_logical(iv, 1)
            hbs[p][pl.ds(c * 16, 16)] = lax.shift_left(iv & 1, 6)

    def g_desc(p):
        return pltpu.make_async_copy(tab_hbm.at[i2s[p]], rows[p], sgs[p])

    def w_desc(h, p):
        return pltpu.make_async_copy(
            tils[p], out_hbm.at[h, :, pl.ds(b0, _L)], sws[p])

    def transpose(p):
        hvecs = [hbs[p][pl.ds(c * 16, 16)] for c in range(8)]

        def per_d(d, carry):
            dv = jnp.full((16,), d, dtype=jnp.int32)
            for c in range(8):
                vals = plsc.load_gather(rows[p], [bvecs[c], hvecs[c] + dv])
                tils[p][d, pl.ds(c * 16, 16)] = vals
            return carry

        lax.fori_loop(0, _D, per_d, 0)

    prep(0, 0)
    g_desc(0).start()
    prep(1, 1)
    g_desc(1).start()

    def body(j, carry):
        for p in range(2):
            h = j * 2 + p
            g_desc(p).wait()

            @pl.when(h >= 2)
            def _():
                w_desc(h - 2, p).wait()

            transpose(p)
            w_desc(h, p).start()

            @pl.when(h + 2 < _H)
            def _():
                prep(h + 2, p)
                g_desc(p).start()

        return carry

    lax.fori_loop(0, _H // 2, body, 0)
    w_desc(_H - 2, 0).wait()
    w_desc(_H - 1, 1).wait()


def kernel(x, table):
    xt = x.astype(jnp.int32).T
    tab2 = jnp.reshape(table, (_V // 2, _D * 2))
    out_t = _gather_t(xt, tab2)
    return jnp.transpose(out_t, (2, 0, 1))
